# baseline (speedup 1.0000x reference)
import jax
import jax.numpy as jnp
from jax.experimental import pallas as pl
from jax.experimental.pallas import tpu as pltpu

_USER = 30000
_ITEM = 20000
_N = _USER + _ITEM
_DIM = 128
_H = 64
_EPS = 1e-5

_B = 1000
_NBU = _USER // _B
_NBI = _ITEM // _B
_NB = _NBU + _NBI


def _body(au_ref, ai_ref, x_ref, w_ref, b_ref,
          lats_ref, gcns_ref,
          adj_s, emb_s,
          sums1_s, sums2_s, atx1u_s, atx1i_s, atx2u_s, atx2i_s,
          csu_s, csi_s):
    p = pl.program_id(0)
    i = pl.program_id(1)
    is_user = i < _NBU

    @pl.when((p == 0) & (i == 0))
    def _zero():
        sums1_s[...] = jnp.zeros_like(sums1_s)
        sums2_s[...] = jnp.zeros_like(sums2_s)
        atx1u_s[...] = jnp.zeros_like(atx1u_s)
        atx1i_s[...] = jnp.zeros_like(atx1i_s)
        atx2u_s[...] = jnp.zeros_like(atx2u_s)
        atx2i_s[...] = jnp.zeros_like(atx2i_s)
        csu_s[...] = jnp.zeros_like(csu_s)
        csi_s[...] = jnp.zeros_like(csi_s)

    @pl.when((p == 0) & (i < _NBU))
    def _park_u():
        adj_s[pl.ds(i * _B, _B), :] = au_ref[...]

    @pl.when((p == 0) & (i < _NBI))
    def _park_i():
        adj_s[pl.ds((_NBU + i) * _B, _B), :] = ai_ref[...]

    a = adj_s[pl.ds(i * _B, _B), :]

    def _dot_tn(m, v):
        return jax.lax.dot_general(m, v, (((0,), (0,)), ((), ())),
                                   preferred_element_type=jnp.float32,
                                   precision=jax.lax.Precision.HIGHEST)

    def _dot(m, v):
        return jnp.dot(m, v, preferred_element_type=jnp.float32,
                       precision=jax.lax.Precision.HIGHEST)

    def _scale_shift(sums_ref, layer, pivoted=False):
        m = sums_ref[0, :] * (1.0 / _N)
        var = sums_ref[1, :] * (1.0 / _N) - m * m
        mean = sums_ref[2, :] + m if pivoted else m
        s = w_ref[layer, :] * jax.lax.rsqrt(var + _EPS)
        t = b_ref[layer, :] - mean * s
        return s, t

    @pl.when(p == 0)
    def _phase0():
        x = x_ref[...]
        emb_s[pl.ds(i * _B, _B), :] = x
        lats_ref[...] = x[None]
        gcns_ref[...] = x[None]
        sums1_s[0:1, :] += jnp.sum(x, axis=0, keepdims=True)
        sums1_s[1:2, :] += jnp.sum(x * x, axis=0, keepdims=True)

        @pl.when(is_user)
        def _():
            atx1u_s[...] += _dot_tn(a, x)
            csu_s[0:1, :] += jnp.sum(a, axis=0, keepdims=True)

        @pl.when(jnp.logical_not(is_user))
        def _():
            atx1i_s[...] += _dot_tn(a, x)
            csi_s[0:1, :] += jnp.sum(a, axis=0, keepdims=True)

    @pl.when(p == 1)
    def _phase1():
        x = emb_s[pl.ds(i * _B, _B), :]
        s, t = _scale_shift(sums1_s, 0)
        e_u = atx1u_s[...] * s[None, :] + csu_s[0:1, :].T * t[None, :]
        e_i = atx1i_s[...] * s[None, :] + csi_s[0:1, :].T * t[None, :]
        e = jnp.where(is_user, e_u, e_i)
        out = _dot(a, e)
        lat = out + x
        gcns_ref[...] = out[None]
        lats_ref[...] = lat[None]
        @pl.when(i == 0)
        def _pivot():
            sums2_s[2:3, :] = jnp.sum(lat, axis=0, keepdims=True) * (1.0 / _B)

        d = lat - sums2_s[2:3, :]
        sums2_s[0:1, :] += jnp.sum(d, axis=0, keepdims=True)
        sums2_s[1:2, :] += jnp.sum(d * d, axis=0, keepdims=True)
        atl = _dot_tn(a, lat)

        @pl.when(is_user)
        def _():
            atx2u_s[...] += atl

        @pl.when(jnp.logical_not(is_user))
        def _():
            atx2i_s[...] += atl

    @pl.when(p == 2)
    def _phase2():
        x = emb_s[pl.ds(i * _B, _B), :]
        s1, t1 = _scale_shift(sums1_s, 0)
        e1_u = atx1u_s[...] * s1[None, :] + csu_s[0:1, :].T * t1[None, :]
        e1_i = atx1i_s[...] * s1[None, :] + csi_s[0:1, :].T * t1[None, :]
        e1 = jnp.where(is_user, e1_u, e1_i)
        lat1 = _dot(a, e1) + x
        s, t = _scale_shift(sums2_s, 1, pivoted=True)
        e_u = atx2u_s[...] * s[None, :] + csu_s[0:1, :].T * t[None, :]
        e_i = atx2i_s[...] * s[None, :] + csi_s[0:1, :].T * t[None, :]
        e = jnp.where(is_user, e_u, e_i)
        out = _dot(a, e)
        gcns_ref[...] = out[None]
        lats_ref[...] = (out + lat1)[None]


def kernel(adj_user, adj_item, embeds, bn_weight, bn_bias):
    grid = (3, _NB)
    lats, gcns = pl.pallas_call(
        _body,
        grid=grid,
        in_specs=[
            pl.BlockSpec((_B, _H),
                         lambda p, i: (jnp.where(p == 0, jnp.minimum(i, _NBU - 1), 0), 0)),
            pl.BlockSpec((_B, _H),
                         lambda p, i: (jnp.where(p == 0, jnp.minimum(i, _NBI - 1), 0), 0)),
            pl.BlockSpec((_B, _DIM),
                         lambda p, i: (jnp.where(p == 0, i, 0), 0)),
            pl.BlockSpec((2, _DIM), lambda p, i: (0, 0)),
            pl.BlockSpec((2, _DIM), lambda p, i: (0, 0)),
        ],
        out_specs=[
            pl.BlockSpec((1, _B, _DIM), lambda p, i: (p, i, 0)),
            pl.BlockSpec((1, _B, _DIM), lambda p, i: (p, i, 0)),
        ],
        out_shape=[
            jax.ShapeDtypeStruct((3, _N, _DIM), jnp.float32),
            jax.ShapeDtypeStruct((3, _N, _DIM), jnp.float32),
        ],
        scratch_shapes=[
            pltpu.VMEM((_N, _H), jnp.float32),
            pltpu.VMEM((_N, _DIM), jnp.float32),
            pltpu.VMEM((8, _DIM), jnp.float32),
            pltpu.VMEM((8, _DIM), jnp.float32),
            pltpu.VMEM((_H, _DIM), jnp.float32),
            pltpu.VMEM((_H, _DIM), jnp.float32),
            pltpu.VMEM((_H, _DIM), jnp.float32),
            pltpu.VMEM((_H, _DIM), jnp.float32),
            pltpu.VMEM((8, _H), jnp.float32),
            pltpu.VMEM((8, _H), jnp.float32),
        ],
    )(adj_user, adj_item, embeds, bn_weight, bn_bias)
    return lats, gcns


# speedup vs baseline: 2.4301x; 2.4301x over previous
import jax
import jax.numpy as jnp
from jax.experimental import pallas as pl
from jax.experimental.pallas import tpu as pltpu

_USER = 30000
_ITEM = 20000
_N = _USER + _ITEM
_DIM = 128
_H = 64
_EPS = 1e-5

_B = 2000
_NBU = _USER // _B
_NBI = _ITEM // _B
_NB = _NBU + _NBI


def _body(au_ref, ai_ref, x_ref, w_ref, b_ref,
          lats_ref, gcns_ref,
          adj_s, emb_s,
          sums1_s, sums2_s, atx1u_s, atx1i_s, atx2u_s, atx2i_s,
          csu_s, csi_s):
    p = pl.program_id(0)
    i = pl.program_id(1)
    is_user = i < _NBU

    @pl.when((p == 0) & (i == 0))
    def _zero():
        sums1_s[...] = jnp.zeros_like(sums1_s)
        sums2_s[...] = jnp.zeros_like(sums2_s)
        atx1u_s[...] = jnp.zeros_like(atx1u_s)
        atx1i_s[...] = jnp.zeros_like(atx1i_s)
        atx2u_s[...] = jnp.zeros_like(atx2u_s)
        atx2i_s[...] = jnp.zeros_like(atx2i_s)
        csu_s[...] = jnp.zeros_like(csu_s)
        csi_s[...] = jnp.zeros_like(csi_s)

    @pl.when((p == 0) & (i < _NBU))
    def _park_u():
        adj_s[pl.ds(i * _B, _B), :] = au_ref[...].astype(jnp.bfloat16)

    @pl.when((p == 0) & (i < _NBI))
    def _park_i():
        adj_s[pl.ds((_NBU + i) * _B, _B), :] = ai_ref[...].astype(jnp.bfloat16)

    a = adj_s[pl.ds(i * _B, _B), :]

    def _dot_tn(m, v):
        return jax.lax.dot_general(m.astype(jnp.bfloat16),
                                   v.astype(jnp.bfloat16),
                                   (((0,), (0,)), ((), ())),
                                   preferred_element_type=jnp.float32)

    def _dot(m, v):
        return jnp.dot(m.astype(jnp.bfloat16), v.astype(jnp.bfloat16),
                       preferred_element_type=jnp.float32)

    def _scale_shift(sums_ref, layer, pivoted=False):
        m = sums_ref[0, :] * (1.0 / _N)
        var = sums_ref[1, :] * (1.0 / _N) - m * m
        mean = sums_ref[2, :] + m if pivoted else m
        s = w_ref[layer, :] * jax.lax.rsqrt(var + _EPS)
        t = b_ref[layer, :] - mean * s
        return s, t

    @pl.when(p == 0)
    def _phase0():
        x = x_ref[...]
        emb_s[pl.ds(i * _B, _B), :] = x
        lats_ref[...] = x[None]
        gcns_ref[...] = x[None]
        sums1_s[0:1, :] += jnp.sum(x, axis=0, keepdims=True)
        sums1_s[1:2, :] += jnp.sum(x * x, axis=0, keepdims=True)

        af = a.astype(jnp.float32)

        @pl.when(is_user)
        def _():
            atx1u_s[...] += _dot_tn(a, x)
            csu_s[0:1, :] += jnp.sum(af, axis=0, keepdims=True)

        @pl.when(jnp.logical_not(is_user))
        def _():
            atx1i_s[...] += _dot_tn(a, x)
            csi_s[0:1, :] += jnp.sum(af, axis=0, keepdims=True)

    @pl.when(p == 1)
    def _phase1():
        x = emb_s[pl.ds(i * _B, _B), :]
        s, t = _scale_shift(sums1_s, 0)
        e_u = atx1u_s[...] * s[None, :] + csu_s[0:1, :].T * t[None, :]
        e_i = atx1i_s[...] * s[None, :] + csi_s[0:1, :].T * t[None, :]
        e = jnp.where(is_user, e_u, e_i)
        out = _dot(a, e)
        lat = out + x
        gcns_ref[...] = out[None]
        lats_ref[...] = lat[None]
        @pl.when(i == 0)
        def _pivot():
            sums2_s[2:3, :] = jnp.sum(lat, axis=0, keepdims=True) * (1.0 / _B)

        d = lat - sums2_s[2:3, :]
        sums2_s[0:1, :] += jnp.sum(d, axis=0, keepdims=True)
        sums2_s[1:2, :] += jnp.sum(d * d, axis=0, keepdims=True)
        atl = _dot_tn(a, lat)

        @pl.when(is_user)
        def _():
            atx2u_s[...] += atl

        @pl.when(jnp.logical_not(is_user))
        def _():
            atx2i_s[...] += atl

    @pl.when(p == 2)
    def _phase2():
        x = emb_s[pl.ds(i * _B, _B), :]
        s1, t1 = _scale_shift(sums1_s, 0)
        e1_u = atx1u_s[...] * s1[None, :] + csu_s[0:1, :].T * t1[None, :]
        e1_i = atx1i_s[...] * s1[None, :] + csi_s[0:1, :].T * t1[None, :]
        e1 = jnp.where(is_user, e1_u, e1_i)
        lat1 = _dot(a, e1) + x
        s, t = _scale_shift(sums2_s, 1, pivoted=True)
        e_u = atx2u_s[...] * s[None, :] + csu_s[0:1, :].T * t[None, :]
        e_i = atx2i_s[...] * s[None, :] + csi_s[0:1, :].T * t[None, :]
        e = jnp.where(is_user, e_u, e_i)
        out = _dot(a, e)
        gcns_ref[...] = out[None]
        lats_ref[...] = (out + lat1)[None]


def kernel(adj_user, adj_item, embeds, bn_weight, bn_bias):
    grid = (3, _NB)
    lats, gcns = pl.pallas_call(
        _body,
        grid=grid,
        in_specs=[
            pl.BlockSpec((_B, _H),
                         lambda p, i: (jnp.where(p == 0, jnp.minimum(i, _NBU - 1), 0), 0)),
            pl.BlockSpec((_B, _H),
                         lambda p, i: (jnp.where(p == 0, jnp.minimum(i, _NBI - 1), 0), 0)),
            pl.BlockSpec((_B, _DIM),
                         lambda p, i: (jnp.where(p == 0, i, 0), 0)),
            pl.BlockSpec((2, _DIM), lambda p, i: (0, 0)),
            pl.BlockSpec((2, _DIM), lambda p, i: (0, 0)),
        ],
        out_specs=[
            pl.BlockSpec((1, _B, _DIM), lambda p, i: (p, i, 0)),
            pl.BlockSpec((1, _B, _DIM), lambda p, i: (p, i, 0)),
        ],
        out_shape=[
            jax.ShapeDtypeStruct((3, _N, _DIM), jnp.float32),
            jax.ShapeDtypeStruct((3, _N, _DIM), jnp.float32),
        ],
        scratch_shapes=[
            pltpu.VMEM((_N, _H), jnp.bfloat16),
            pltpu.VMEM((_N, _DIM), jnp.float32),
            pltpu.VMEM((8, _DIM), jnp.float32),
            pltpu.VMEM((8, _DIM), jnp.float32),
            pltpu.VMEM((_H, _DIM), jnp.float32),
            pltpu.VMEM((_H, _DIM), jnp.float32),
            pltpu.VMEM((_H, _DIM), jnp.float32),
            pltpu.VMEM((_H, _DIM), jnp.float32),
            pltpu.VMEM((8, _H), jnp.float32),
            pltpu.VMEM((8, _H), jnp.float32),
        ],
    )(adj_user, adj_item, embeds, bn_weight, bn_bias)
    return lats, gcns


# speedup vs baseline: 2.4754x; 1.0186x over previous
import jax
import jax.numpy as jnp
from jax.experimental import pallas as pl
from jax.experimental.pallas import tpu as pltpu

_USER = 30000
_ITEM = 20000
_N = _USER + _ITEM
_DIM = 128
_H = 64
_EPS = 1e-5

_B = 2000
_NBU = _USER // _B
_NBI = _ITEM // _B
_NB = _NBU + _NBI


def _body(au_ref, ai_ref, x_ref, w_ref, b_ref,
          lats_ref, gcns_ref,
          adj_s, emb_s,
          sums1_s, sums2_s, atx1u_s, atx1i_s, atx2u_s, atx2i_s,
          csu_s, csi_s, e1_s, e2_s):
    p = pl.program_id(0)
    i = pl.program_id(1)
    is_user = i < _NBU

    @pl.when((p == 0) & (i == 0))
    def _zero():
        sums1_s[...] = jnp.zeros_like(sums1_s)
        sums2_s[...] = jnp.zeros_like(sums2_s)
        atx1u_s[...] = jnp.zeros_like(atx1u_s)
        atx1i_s[...] = jnp.zeros_like(atx1i_s)
        atx2u_s[...] = jnp.zeros_like(atx2u_s)
        atx2i_s[...] = jnp.zeros_like(atx2i_s)
        csu_s[...] = jnp.zeros_like(csu_s)
        csi_s[...] = jnp.zeros_like(csi_s)

    @pl.when((p == 0) & (i < _NBU))
    def _park_u():
        au = au_ref[...]
        adj_s[pl.ds(i * _B, _B), :] = au.astype(jnp.bfloat16)
        csu_s[0:1, :] += jnp.sum(au, axis=0, keepdims=True)

    @pl.when((p == 0) & (i < _NBI))
    def _park_i():
        ai = ai_ref[...]
        adj_s[pl.ds((_NBU + i) * _B, _B), :] = ai.astype(jnp.bfloat16)
        csi_s[0:1, :] += jnp.sum(ai, axis=0, keepdims=True)

    a = adj_s[pl.ds(i * _B, _B), :]

    def _dot_tn(m, v):
        return jax.lax.dot_general(m.astype(jnp.bfloat16),
                                   v.astype(jnp.bfloat16),
                                   (((0,), (0,)), ((), ())),
                                   preferred_element_type=jnp.float32)

    def _dot(m, v):
        return jnp.dot(m.astype(jnp.bfloat16), v.astype(jnp.bfloat16),
                       preferred_element_type=jnp.float32)

    def _scale_shift(sums_ref, layer, pivoted=False):
        m = sums_ref[0, :] * (1.0 / _N)
        var = sums_ref[1, :] * (1.0 / _N) - m * m
        mean = sums_ref[2, :] + m if pivoted else m
        s = w_ref[layer, :] * jax.lax.rsqrt(var + _EPS)
        t = b_ref[layer, :] - mean * s
        return s, t

    @pl.when(p == 0)
    def _phase0():
        x = x_ref[...]
        emb_s[pl.ds(i * _B, _B), :] = x
        lats_ref[...] = x[None]
        gcns_ref[...] = x[None]
        sums1_s[0:1, :] += jnp.sum(x, axis=0, keepdims=True)
        sums1_s[1:2, :] += jnp.sum(x * x, axis=0, keepdims=True)

        @pl.when(is_user)
        def _():
            atx1u_s[...] += _dot_tn(a, x)

        @pl.when(jnp.logical_not(is_user))
        def _():
            atx1i_s[...] += _dot_tn(a, x)

    idx = jnp.where(is_user, 0, 1)

    @pl.when(p == 1)
    def _phase1():
        @pl.when(i == 0)
        def _build_e1():
            s, t = _scale_shift(sums1_s, 0)
            e_u = atx1u_s[...] * s[None, :] + csu_s[0:1, :].T * t[None, :]
            e_i = atx1i_s[...] * s[None, :] + csi_s[0:1, :].T * t[None, :]
            e1_s[0] = e_u.astype(jnp.bfloat16)
            e1_s[1] = e_i.astype(jnp.bfloat16)

        x = emb_s[pl.ds(i * _B, _B), :]
        out = _dot(a, e1_s[idx])
        lat = out + x
        gcns_ref[...] = out[None]
        lats_ref[...] = lat[None]
        @pl.when(i == 0)
        def _pivot():
            sums2_s[2:3, :] = jnp.sum(lat, axis=0, keepdims=True) * (1.0 / _B)

        d = lat - sums2_s[2:3, :]
        sums2_s[0:1, :] += jnp.sum(d, axis=0, keepdims=True)
        sums2_s[1:2, :] += jnp.sum(d * d, axis=0, keepdims=True)
        atl = _dot_tn(a, lat)

        @pl.when(is_user)
        def _():
            atx2u_s[...] += atl

        @pl.when(jnp.logical_not(is_user))
        def _():
            atx2i_s[...] += atl

    @pl.when(p == 2)
    def _phase2():
        @pl.when(i == 0)
        def _build_e2():
            s, t = _scale_shift(sums2_s, 1, pivoted=True)
            e_u = atx2u_s[...] * s[None, :] + csu_s[0:1, :].T * t[None, :]
            e_i = atx2i_s[...] * s[None, :] + csi_s[0:1, :].T * t[None, :]
            e2_s[0] = e_u.astype(jnp.bfloat16)
            e2_s[1] = e_i.astype(jnp.bfloat16)

        x = emb_s[pl.ds(i * _B, _B), :]
        lat1 = _dot(a, e1_s[idx]) + x
        out = _dot(a, e2_s[idx])
        gcns_ref[...] = out[None]
        lats_ref[...] = (out + lat1)[None]


def kernel(adj_user, adj_item, embeds, bn_weight, bn_bias):
    grid = (3, _NB)
    lats, gcns = pl.pallas_call(
        _body,
        grid=grid,
        in_specs=[
            pl.BlockSpec((_B, _H),
                         lambda p, i: (jnp.where(p == 0, jnp.minimum(i, _NBU - 1), 0), 0)),
            pl.BlockSpec((_B, _H),
                         lambda p, i: (jnp.where(p == 0, jnp.minimum(i, _NBI - 1), 0), 0)),
            pl.BlockSpec((_B, _DIM),
                         lambda p, i: (jnp.where(p == 0, i, 0), 0)),
            pl.BlockSpec((2, _DIM), lambda p, i: (0, 0)),
            pl.BlockSpec((2, _DIM), lambda p, i: (0, 0)),
        ],
        out_specs=[
            pl.BlockSpec((1, _B, _DIM), lambda p, i: (p, i, 0)),
            pl.BlockSpec((1, _B, _DIM), lambda p, i: (p, i, 0)),
        ],
        out_shape=[
            jax.ShapeDtypeStruct((3, _N, _DIM), jnp.float32),
            jax.ShapeDtypeStruct((3, _N, _DIM), jnp.float32),
        ],
        scratch_shapes=[
            pltpu.VMEM((_N, _H), jnp.bfloat16),
            pltpu.VMEM((_N, _DIM), jnp.float32),
            pltpu.VMEM((8, _DIM), jnp.float32),
            pltpu.VMEM((8, _DIM), jnp.float32),
            pltpu.VMEM((_H, _DIM), jnp.float32),
            pltpu.VMEM((_H, _DIM), jnp.float32),
            pltpu.VMEM((_H, _DIM), jnp.float32),
            pltpu.VMEM((_H, _DIM), jnp.float32),
            pltpu.VMEM((8, _H), jnp.float32),
            pltpu.VMEM((8, _H), jnp.float32),
            pltpu.VMEM((2, _H, _DIM), jnp.bfloat16),
            pltpu.VMEM((2, _H, _DIM), jnp.bfloat16),
        ],
    )(adj_user, adj_item, embeds, bn_weight, bn_bias)
    return lats, gcns


# speedup vs baseline: 2.5435x; 1.0275x over previous
import jax
import jax.numpy as jnp
from jax.experimental import pallas as pl
from jax.experimental.pallas import tpu as pltpu

_USER = 30000
_ITEM = 20000
_N = _USER + _ITEM
_DIM = 128
_H = 64
_EPS = 1e-5

_B = 2000
_NBU = _USER // _B
_NBI = _ITEM // _B
_NB = _NBU + _NBI


def _body(au_ref, ai_ref, x_ref, w_ref, b_ref,
          lats_ref, gcns_ref,
          adj_s, emb_s,
          sums1_s, sums2_s, atx1u_s, atx1i_s, atx2u_s, atx2i_s,
          csu_s, csi_s, e1_s, e2_s):
    p = pl.program_id(0)
    i = pl.program_id(1)
    is_user = i < _NBU

    @pl.when((p == 0) & (i == 0))
    def _zero():
        sums1_s[...] = jnp.zeros_like(sums1_s)
        sums2_s[...] = jnp.zeros_like(sums2_s)
        atx1u_s[...] = jnp.zeros_like(atx1u_s)
        atx1i_s[...] = jnp.zeros_like(atx1i_s)
        atx2u_s[...] = jnp.zeros_like(atx2u_s)
        atx2i_s[...] = jnp.zeros_like(atx2i_s)
        csu_s[...] = jnp.zeros_like(csu_s)
        csi_s[...] = jnp.zeros_like(csi_s)

    @pl.when((p == 0) & (i < _NBU))
    def _park_u():
        au = au_ref[...]
        adj_s[pl.ds(i * _B, _B), :] = au.astype(jnp.bfloat16)
        csu_s[0:1, :] += jnp.sum(au, axis=0, keepdims=True)

    @pl.when((p == 0) & (i < _NBI))
    def _park_i():
        ai = ai_ref[...]
        adj_s[pl.ds((_NBU + i) * _B, _B), :] = ai.astype(jnp.bfloat16)
        csi_s[0:1, :] += jnp.sum(ai, axis=0, keepdims=True)

    a = adj_s[pl.ds(i * _B, _B), :]

    def _dot_tn(m, v):
        return jax.lax.dot_general(m.astype(jnp.bfloat16),
                                   v.astype(jnp.bfloat16),
                                   (((0,), (0,)), ((), ())),
                                   preferred_element_type=jnp.float32)

    def _dot(m, v):
        return jnp.dot(m.astype(jnp.bfloat16), v.astype(jnp.bfloat16),
                       preferred_element_type=jnp.float32)

    def _scale_shift(sums_ref, layer, pivoted=False):
        m = sums_ref[0, :] * (1.0 / _N)
        var = sums_ref[1, :] * (1.0 / _N) - m * m
        mean = sums_ref[2, :] + m if pivoted else m
        s = w_ref[layer, :] * jax.lax.rsqrt(var + _EPS)
        t = b_ref[layer, :] - mean * s
        return s, t

    @pl.when(p == 0)
    def _phase0():
        x = x_ref[...]
        emb_s[pl.ds(i * _B, _B), :] = x
        lats_ref[...] = x[None]
        gcns_ref[...] = x[None]
        sums1_s[0:1, :] += jnp.sum(x, axis=0, keepdims=True)
        sums1_s[1:2, :] += jnp.sum(x * x, axis=0, keepdims=True)

        @pl.when(is_user)
        def _():
            atx1u_s[...] += _dot_tn(a, x)

        @pl.when(jnp.logical_not(is_user))
        def _():
            atx1i_s[...] += _dot_tn(a, x)

    idx = jnp.where(is_user, 0, 1)

    @pl.when(p == 1)
    def _phase1():
        @pl.when(i == 0)
        def _build_e1():
            s, t = _scale_shift(sums1_s, 0)
            e_u = atx1u_s[...] * s[None, :] + csu_s[0:1, :].T * t[None, :]
            e_i = atx1i_s[...] * s[None, :] + csi_s[0:1, :].T * t[None, :]
            e1_s[0] = e_u.astype(jnp.bfloat16)
            e1_s[1] = e_i.astype(jnp.bfloat16)

        x = emb_s[pl.ds(i * _B, _B), :]
        out = _dot(a, e1_s[idx])
        lat = out + x
        gcns_ref[...] = out[None]
        lats_ref[...] = lat[None]
        emb_s[pl.ds(i * _B, _B), :] = lat
        @pl.when(i == 0)
        def _pivot():
            sums2_s[2:3, :] = jnp.sum(lat, axis=0, keepdims=True) * (1.0 / _B)

        d = lat - sums2_s[2:3, :]
        sums2_s[0:1, :] += jnp.sum(d, axis=0, keepdims=True)
        sums2_s[1:2, :] += jnp.sum(d * d, axis=0, keepdims=True)
        atl = _dot_tn(a, lat)

        @pl.when(is_user)
        def _():
            atx2u_s[...] += atl

        @pl.when(jnp.logical_not(is_user))
        def _():
            atx2i_s[...] += atl

    @pl.when(p == 2)
    def _phase2():
        @pl.when(i == 0)
        def _build_e2():
            s, t = _scale_shift(sums2_s, 1, pivoted=True)
            e_u = atx2u_s[...] * s[None, :] + csu_s[0:1, :].T * t[None, :]
            e_i = atx2i_s[...] * s[None, :] + csi_s[0:1, :].T * t[None, :]
            e2_s[0] = e_u.astype(jnp.bfloat16)
            e2_s[1] = e_i.astype(jnp.bfloat16)

        lat1 = emb_s[pl.ds(i * _B, _B), :]
        out = _dot(a, e2_s[idx])
        gcns_ref[...] = out[None]
        lats_ref[...] = (out + lat1)[None]


def kernel(adj_user, adj_item, embeds, bn_weight, bn_bias):
    grid = (3, _NB)
    lats, gcns = pl.pallas_call(
        _body,
        grid=grid,
        in_specs=[
            pl.BlockSpec((_B, _H),
                         lambda p, i: (jnp.where(p == 0, jnp.minimum(i, _NBU - 1), 0), 0)),
            pl.BlockSpec((_B, _H),
                         lambda p, i: (jnp.where(p == 0, jnp.minimum(i, _NBI - 1), 0), 0)),
            pl.BlockSpec((_B, _DIM),
                         lambda p, i: (jnp.where(p == 0, i, 0), 0)),
            pl.BlockSpec((2, _DIM), lambda p, i: (0, 0)),
            pl.BlockSpec((2, _DIM), lambda p, i: (0, 0)),
        ],
        out_specs=[
            pl.BlockSpec((1, _B, _DIM), lambda p, i: (p, i, 0)),
            pl.BlockSpec((1, _B, _DIM), lambda p, i: (p, i, 0)),
        ],
        out_shape=[
            jax.ShapeDtypeStruct((3, _N, _DIM), jnp.float32),
            jax.ShapeDtypeStruct((3, _N, _DIM), jnp.float32),
        ],
        scratch_shapes=[
            pltpu.VMEM((_N, _H), jnp.bfloat16),
            pltpu.VMEM((_N, _DIM), jnp.float32),
            pltpu.VMEM((8, _DIM), jnp.float32),
            pltpu.VMEM((8, _DIM), jnp.float32),
            pltpu.VMEM((_H, _DIM), jnp.float32),
            pltpu.VMEM((_H, _DIM), jnp.float32),
            pltpu.VMEM((_H, _DIM), jnp.float32),
            pltpu.VMEM((_H, _DIM), jnp.float32),
            pltpu.VMEM((8, _H), jnp.float32),
            pltpu.VMEM((8, _H), jnp.float32),
            pltpu.VMEM((2, _H, _DIM), jnp.bfloat16),
            pltpu.VMEM((2, _H, _DIM), jnp.bfloat16),
        ],
    )(adj_user, adj_item, embeds, bn_weight, bn_bias)
    return lats, gcns
